# Initial kernel scaffold; baseline (speedup 1.0000x reference)
import functools

import jax
import jax.numpy as jnp
from jax import lax
from jax.experimental import pallas as pl
from jax.experimental.pallas import tpu as pltpu
from jax.experimental.pallas import tpu_sc as plsc

N = 16777216
VOCAB = 100000

NC = 2
NS = 16
NW = NC * NS
L = 16

VOCABP = 100352
S = VOCABP // NW
CHUNK = 4096
PER_W = N // NW
PAIRS = PER_W // (2 * CHUNK)

CCHUNK = VOCABP // 16
BUF = CCHUNK + 8
BUFA = 6304
OUT_PAD = 106496
MAXZ = OUT_PAD // BUF + 1

_mesh = plsc.VectorSubcoreMesh(
    core_axis_name="c", subcore_axis_name="s", num_cores=NC, num_subcores=NS
)


def _worker_id():
    return lax.axis_index("s") * NC + lax.axis_index("c")


@functools.partial(
    pl.kernel,
    out_type=jax.ShapeDtypeStruct((NW * VOCABP,), jnp.int32),
    mesh=_mesh,
    scratch_types=[
        pltpu.VMEM((VOCABP,), jnp.int32),
        pltpu.VMEM((CHUNK,), jnp.int32),
        pltpu.VMEM((CHUNK,), jnp.int32),
        pltpu.SemaphoreType.DMA,
        pltpu.SemaphoreType.DMA,
    ],
)
def _hist_kernel(x_hbm, part_hbm, hist, buf0, buf1, sem0, sem1):
    wid = _worker_id()
    base = wid * PER_W
    zeros16 = jnp.zeros((L,), jnp.int32)
    ones16 = jnp.ones((L,), jnp.int32)

    def zbody(i, carry):
        hist[pl.ds(i * L, L)] = zeros16
        return carry

    lax.fori_loop(0, VOCABP // L, zbody, 0, unroll=4)

    def copy_in(chunk_idx, buf, sem):
        return pltpu.make_async_copy(
            x_hbm.at[pl.ds(base + chunk_idx * CHUNK, CHUNK)], buf, sem
        )

    def process(buf):
        def pbody(j, carry):
            idx = buf[pl.ds(j * L, L)]
            plsc.addupdate_scatter(hist, [idx], ones16)
            return carry

        lax.fori_loop(0, CHUNK // L, pbody, 0, unroll=4)

    copy_in(0, buf0, sem0).start()

    def body(p, carry):
        copy_in(2 * p + 1, buf1, sem1).start()
        copy_in(2 * p, buf0, sem0).wait()
        process(buf0)

        @pl.when(p < PAIRS - 1)
        def _():
            copy_in(2 * p + 2, buf0, sem0).start()

        copy_in(2 * p + 1, buf1, sem1).wait()
        process(buf1)
        return carry

    lax.fori_loop(0, PAIRS, body, 0)

    pltpu.sync_copy(hist, part_hbm.at[pl.ds(wid * VOCABP, VOCABP)])


@functools.partial(
    pl.kernel,
    out_type=jax.ShapeDtypeStruct((VOCABP,), jnp.int32),
    mesh=_mesh,
    scratch_types=[
        pltpu.VMEM((S,), jnp.int32),
        pltpu.VMEM((S,), jnp.int32),
    ],
)
def _merge_kernel(part_hbm, hist_hbm, acc, tmp):
    wid = _worker_id()
    col = wid * S
    pltpu.sync_copy(part_hbm.at[pl.ds(col, S)], acc)

    def rbody(r, carry):
        pltpu.sync_copy(part_hbm.at[pl.ds(r * VOCABP + col, S)], tmp)

        def jbody(j, c2):
            sl = pl.ds(j * L, L)
            acc[sl] = acc[sl] + tmp[sl]
            return c2

        lax.fori_loop(0, S // L, jbody, 0, unroll=4)
        return carry

    lax.fori_loop(1, NW, rbody, 0)
    pltpu.sync_copy(acc, hist_hbm.at[pl.ds(col, S)])


@functools.partial(
    pl.kernel,
    out_type=(
        jax.ShapeDtypeStruct((OUT_PAD,), jnp.int32),
        jax.ShapeDtypeStruct((OUT_PAD,), jnp.int32),
    ),
    mesh=_mesh,
    scratch_types=[
        pltpu.VMEM((CCHUNK,), jnp.int32),
        pltpu.VMEM((BUFA,), jnp.int32),
        pltpu.VMEM((BUFA,), jnp.int32),
    ],
)
def _compact_kernel(hist_hbm, val_hbm, cnt_hbm, hchunk, vbuf, cbuf):
    cid = lax.axis_index("c")
    sid = lax.axis_index("s")

    @pl.when(jnp.logical_and(cid == 0, sid == 0))
    def _():
        iota = lax.iota(jnp.int32, L)
        zeros16 = jnp.zeros((L,), jnp.int32)

        def zero_bufs(i, carry):
            vbuf[pl.ds(i * L, L)] = zeros16
            cbuf[pl.ds(i * L, L)] = zeros16
            return carry

        lax.fori_loop(0, BUFA // L, zero_bufs, 0, unroll=4)

        def chunk_body(ci, carry):
            f, g = carry
            pltpu.sync_copy(hist_hbm.at[pl.ds(ci * CCHUNK, CCHUNK)], hchunk)

            def step(j, fc):
                h = hchunk[pl.ds(j * L, L)]
                m = h > 0
                vals = (ci * CCHUNK + j * L) + iota
                plsc.store_compressed(vbuf.at[pl.ds(fc, L)], vals, mask=m)
                plsc.store_compressed(cbuf.at[pl.ds(fc, L)], h, mask=m)
                pc = jnp.sum(jnp.where(m, 1, 0).astype(jnp.int32))
                return fc + pc

            f = lax.fori_loop(0, CCHUNK // L, step, f)

            rem = jnp.bitwise_and(f, 7)
            k = f - rem
            tailv = vbuf[pl.ds(k, L)]
            tailc = cbuf[pl.ds(k, L)]
            tailv = jnp.where(iota < rem, tailv, 0)
            tailc = jnp.where(iota < rem, tailc, 0)

            pltpu.sync_copy(vbuf.at[pl.ds(0, BUF)], val_hbm.at[pl.ds(g, BUF)])
            pltpu.sync_copy(cbuf.at[pl.ds(0, BUF)], cnt_hbm.at[pl.ds(g, BUF)])

            lax.fori_loop(0, BUFA // L, zero_bufs, 0, unroll=4)
            vbuf[pl.ds(0, L)] = tailv
            cbuf[pl.ds(0, L)] = tailc
            return (rem, g + k)

        f, g = lax.fori_loop(0, VOCABP // CCHUNK, chunk_body, (0, 0))

        pltpu.sync_copy(vbuf.at[pl.ds(0, BUF)], val_hbm.at[pl.ds(g, BUF)])
        pltpu.sync_copy(cbuf.at[pl.ds(0, BUF)], cnt_hbm.at[pl.ds(g, BUF)])

        lax.fori_loop(0, BUFA // L, zero_bufs, 0, unroll=4)

        def ztail(i, carry):
            off = jnp.minimum(g + BUF + i * BUF, OUT_PAD - BUF)
            pltpu.sync_copy(vbuf.at[pl.ds(0, BUF)], val_hbm.at[pl.ds(off, BUF)])
            pltpu.sync_copy(cbuf.at[pl.ds(0, BUF)], cnt_hbm.at[pl.ds(off, BUF)])
            return carry

        lax.fori_loop(0, MAXZ, ztail, 0)


def kernel(input_tensor, output_tensor, count_tensor):
    del output_tensor, count_tensor
    x = input_tensor.astype(jnp.int32)
    partials = _hist_kernel(x)
    hist = _merge_kernel(partials)
    values, counts = _compact_kernel(hist)
    return (
        values[:VOCAB].astype(jnp.int64),
        counts[:VOCAB].astype(jnp.int64),
    )


# speedup vs baseline: 3541.7954x; 3541.7954x over previous
import functools

import jax
import jax.numpy as jnp
from jax import lax
from jax.experimental import pallas as pl
from jax.experimental.pallas import tpu as pltpu
from jax.experimental.pallas import tpu_sc as plsc

N = 16777216
VOCAB = 100000

NC = 2
NS = 16
NW = NC * NS
L = 16

VOCABP = 100352
S = VOCABP // NW
CHUNK = 8192
PER_W = N // NW
PAIRS = PER_W // (2 * CHUNK)

CCHUNK = VOCABP // 16
BUF = CCHUNK + 8
BUFA = 6304
OUT_PAD = 106496
MAXZ = OUT_PAD // BUF + 1

_mesh = plsc.VectorSubcoreMesh(
    core_axis_name="c", subcore_axis_name="s", num_cores=NC, num_subcores=NS
)

_params = pltpu.CompilerParams(needs_layout_passes=False)


def _i32(v):
    return jnp.int32(v)


def _fori(lo, hi, body, init):
    return lax.fori_loop(jnp.int32(lo), jnp.int32(hi), body, init)


def _worker_id():
    return lax.axis_index("s") * NC + lax.axis_index("c")


@functools.partial(
    pl.kernel,
    out_type=jax.ShapeDtypeStruct((NW * VOCABP,), jnp.int32),
    mesh=_mesh,
    compiler_params=_params,
    scratch_types=[
        pltpu.VMEM((VOCABP,), jnp.int32),
        pltpu.VMEM((CHUNK,), jnp.int32),
        pltpu.VMEM((CHUNK,), jnp.int32),
        pltpu.SemaphoreType.DMA,
        pltpu.SemaphoreType.DMA,
    ],
)
def _hist_kernel(x_hbm, part_hbm, hist, buf0, buf1, sem0, sem1):
    wid = _worker_id()
    base = wid * _i32(PER_W)
    zeros16 = jnp.zeros((L,), jnp.int32)
    ones16 = jnp.ones((L,), jnp.int32)

    def zbody(i, carry):
        base8 = i * _i32(8 * L)
        for u in range(8):
            hist[pl.ds(base8 + u * L, L)] = zeros16
        return carry

    _fori(0, VOCABP // (8 * L), zbody, jnp.int32(0))

    def copy_in(chunk_idx, buf, sem):
        return pltpu.make_async_copy(
            x_hbm.at[
                pl.ds(
                    pl.multiple_of(base + chunk_idx * _i32(CHUNK), 8),
                    CHUNK,
                )
            ],
            buf,
            sem,
        )

    def process(buf):
        def pbody(j, carry):
            base8 = j * _i32(8 * L)
            for u in range(8):
                w = buf[pl.ds(base8 + u * L, L)]
                plsc.addupdate_scatter(hist, [w], ones16)
            return carry

        _fori(0, CHUNK // (8 * L), pbody, jnp.int32(0))

    copy_in(0, buf0, sem0).start()

    def body(p, carry):
        copy_in(2 * p + 1, buf1, sem1).start()
        copy_in(2 * p, buf0, sem0).wait()
        process(buf0)

        @pl.when(p < PAIRS - 1)
        def _():
            copy_in(2 * p + 2, buf0, sem0).start()

        copy_in(2 * p + 1, buf1, sem1).wait()
        process(buf1)
        return carry

    _fori(0, PAIRS, body, jnp.int32(0))

    pltpu.sync_copy(hist, part_hbm.at[pl.ds(pl.multiple_of(wid * _i32(VOCABP), 8), VOCABP)])


@functools.partial(
    pl.kernel,
    out_type=jax.ShapeDtypeStruct((VOCABP,), jnp.int32),
    mesh=_mesh,
    compiler_params=_params,
    scratch_types=[
        pltpu.VMEM((S,), jnp.int32),
        pltpu.VMEM((S,), jnp.int32),
    ],
)
def _merge_kernel(part_hbm, hist_hbm, acc, tmp):
    wid = _worker_id()
    col = pl.multiple_of(wid * _i32(S), 8)
    pltpu.sync_copy(part_hbm.at[pl.ds(col, S)], acc)

    def rbody(r, carry):
        pltpu.sync_copy(part_hbm.at[pl.ds(pl.multiple_of(r * _i32(VOCABP) + col, 8), S)], tmp)

        def jbody(j, c2):
            base4 = j * _i32(4 * L)
            for u in range(4):
                sl = pl.ds(base4 + u * L, L)
                acc[sl] = acc[sl] + tmp[sl]
            return c2

        _fori(0, S // (4 * L), jbody, jnp.int32(0))
        return carry

    _fori(1, NW, rbody, jnp.int32(0))
    pltpu.sync_copy(acc, hist_hbm.at[pl.ds(col, S)])


@functools.partial(
    pl.kernel,
    out_type=(
        jax.ShapeDtypeStruct((OUT_PAD,), jnp.int32),
        jax.ShapeDtypeStruct((OUT_PAD,), jnp.int32),
    ),
    mesh=_mesh,
    compiler_params=_params,
    scratch_types=[
        pltpu.VMEM((CCHUNK,), jnp.int32),
        pltpu.VMEM((BUFA,), jnp.int32),
        pltpu.VMEM((BUFA,), jnp.int32),
    ],
)
def _compact_kernel(hist_hbm, val_hbm, cnt_hbm, hchunk, vbuf, cbuf):
    cid = lax.axis_index("c")
    sid = lax.axis_index("s")

    @pl.when(jnp.logical_and(cid == 0, sid == 0))
    def _():
        iota = lax.iota(jnp.int32, L)
        zeros16 = jnp.zeros((L,), jnp.int32)

        def zero_bufs(i, carry):
            base2 = i * _i32(2 * L)
            for u in range(2):
                vbuf[pl.ds(base2 + u * L, L)] = zeros16
                cbuf[pl.ds(base2 + u * L, L)] = zeros16
            return carry

        _fori(0, BUFA // (2 * L), zero_bufs, jnp.int32(0))

        def chunk_body(ci, carry):
            f, g = carry
            pltpu.sync_copy(hist_hbm.at[pl.ds(pl.multiple_of(ci * _i32(CCHUNK), 8), CCHUNK)], hchunk)

            def step(j, sc):
                fc, vals = sc
                for u in range(2):
                    h = hchunk[pl.ds(j * _i32(2 * L) + u * L, L)]
                    m = h > 0
                    plsc.store_compressed(vbuf.at[pl.ds(fc, L)], vals, mask=m)
                    plsc.store_compressed(cbuf.at[pl.ds(fc, L)], h, mask=m)
                    pc = plsc.all_reduce_population_count(m)[0]
                    fc = fc + pc
                    vals = vals + _i32(L)
                return (fc, vals)

            vals0 = ci * _i32(CCHUNK) + iota
            f, _unused = _fori(0, CCHUNK // (2 * L), step, (f, vals0))

            rem = jnp.bitwise_and(f, 7)
            k = f - rem
            tailv = vbuf[pl.ds(k, L)]
            tailc = cbuf[pl.ds(k, L)]
            tailv = jnp.where(iota < rem, tailv, jnp.int32(0))
            tailc = jnp.where(iota < rem, tailc, jnp.int32(0))

            pltpu.sync_copy(vbuf.at[pl.ds(0, BUF)], val_hbm.at[pl.ds(pl.multiple_of(g, 8), BUF)])
            pltpu.sync_copy(cbuf.at[pl.ds(0, BUF)], cnt_hbm.at[pl.ds(pl.multiple_of(g, 8), BUF)])

            _fori(0, BUFA // (2 * L), zero_bufs, jnp.int32(0))
            vbuf[pl.ds(0, L)] = tailv
            cbuf[pl.ds(0, L)] = tailc
            return (rem, g + k)

        f, g = _fori(0, VOCABP // CCHUNK, chunk_body, (jnp.int32(0), jnp.int32(0)))

        pltpu.sync_copy(vbuf.at[pl.ds(0, BUF)], val_hbm.at[pl.ds(pl.multiple_of(g, 8), BUF)])
        pltpu.sync_copy(cbuf.at[pl.ds(0, BUF)], cnt_hbm.at[pl.ds(pl.multiple_of(g, 8), BUF)])

        _fori(0, BUFA // (2 * L), zero_bufs, jnp.int32(0))

        def ztail(i, carry):
            off = jnp.minimum(g + _i32(BUF) + i * _i32(BUF), _i32(OUT_PAD - BUF))
            pltpu.sync_copy(vbuf.at[pl.ds(0, BUF)], val_hbm.at[pl.ds(pl.multiple_of(off, 8), BUF)])
            pltpu.sync_copy(cbuf.at[pl.ds(0, BUF)], cnt_hbm.at[pl.ds(pl.multiple_of(off, 8), BUF)])
            return carry

        _fori(0, MAXZ, ztail, jnp.int32(0))


def kernel(input_tensor, output_tensor, count_tensor):
    del output_tensor, count_tensor
    x = input_tensor.astype(jnp.int32)
    partials = _hist_kernel(x)
    hist = _merge_kernel(partials)
    values, counts = _compact_kernel(hist)
    return (
        values[:VOCAB].astype(jnp.int64),
        counts[:VOCAB].astype(jnp.int64),
    )


# speedup vs baseline: 4490.3657x; 1.2678x over previous
import functools

import jax
import jax.numpy as jnp
from jax import lax
from jax.experimental import pallas as pl
from jax.experimental.pallas import tpu as pltpu
from jax.experimental.pallas import tpu_sc as plsc

N = 16777216
VOCAB = 100000

NC = 2
NS = 16
NW = NC * NS
L = 16

VOCABP = 100352
S = VOCABP // NW
CHUNK = 8192
PER_W = N // NW
PAIRS = PER_W // (2 * CHUNK)

CCHUNK = VOCABP // 16
BUF = CCHUNK + 8
BUFA = 6304
OUT_PAD = 106496
MAXZ = OUT_PAD // BUF + 1

_mesh = plsc.VectorSubcoreMesh(
    core_axis_name="c", subcore_axis_name="s", num_cores=NC, num_subcores=NS
)

_params = pltpu.CompilerParams(needs_layout_passes=False)


def _i32(v):
    return jnp.int32(v)


def _fori(lo, hi, body, init):
    return lax.fori_loop(jnp.int32(lo), jnp.int32(hi), body, init)


def _worker_id():
    return lax.axis_index("s") * NC + lax.axis_index("c")


@functools.partial(
    pl.kernel,
    out_type=jax.ShapeDtypeStruct((NW * VOCABP,), jnp.int32),
    mesh=_mesh,
    compiler_params=_params,
    scratch_types=[
        pltpu.VMEM((VOCABP,), jnp.int32),
        pltpu.VMEM((CHUNK,), jnp.uint32),
        pltpu.VMEM((CHUNK,), jnp.uint32),
        pltpu.SemaphoreType.DMA,
        pltpu.SemaphoreType.DMA,
    ],
)
def _hist_kernel(x_hbm, part_hbm, hist, buf0, buf1, sem0, sem1):
    wid = _worker_id()
    base = wid * _i32(PER_W)
    zeros16 = jnp.zeros((L,), jnp.int32)
    ones16 = jnp.ones((L,), jnp.int32)

    def zbody(i, carry):
        base8 = i * _i32(8 * L)
        for u in range(8):
            hist[pl.ds(base8 + u * L, L)] = zeros16
        return carry

    _fori(0, VOCABP // (8 * L), zbody, jnp.int32(0))

    def copy_in(chunk_idx, buf, sem):
        return pltpu.make_async_copy(
            x_hbm.at[
                pl.ds(
                    pl.multiple_of(base + chunk_idx * _i32(CHUNK), 8),
                    CHUNK,
                )
            ],
            buf,
            sem,
        )

    def process(buf):
        @functools.partial(plsc.parallel_loop, 0, CHUNK // (8 * L), unroll=2)
        def _(j):
            base8 = j * _i32(8 * L)
            for u in range(8):
                w = plsc.bitcast(buf[pl.ds(base8 + u * L, L)], jnp.int32)
                plsc.addupdate_scatter(hist, [w], ones16)

    copy_in(0, buf0, sem0).start()

    def body(p, carry):
        copy_in(2 * p + 1, buf1, sem1).start()
        copy_in(2 * p, buf0, sem0).wait()
        process(buf0)

        @pl.when(p < PAIRS - 1)
        def _():
            copy_in(2 * p + 2, buf0, sem0).start()

        copy_in(2 * p + 1, buf1, sem1).wait()
        process(buf1)
        return carry

    _fori(0, PAIRS, body, jnp.int32(0))

    pltpu.sync_copy(hist, part_hbm.at[pl.ds(pl.multiple_of(wid * _i32(VOCABP), 8), VOCABP)])


@functools.partial(
    pl.kernel,
    out_type=jax.ShapeDtypeStruct((VOCABP,), jnp.int32),
    mesh=_mesh,
    compiler_params=_params,
    scratch_types=[
        pltpu.VMEM((S,), jnp.int32),
        pltpu.VMEM((S,), jnp.int32),
    ],
)
def _merge_kernel(part_hbm, hist_hbm, acc, tmp):
    wid = _worker_id()
    col = pl.multiple_of(wid * _i32(S), 8)
    pltpu.sync_copy(part_hbm.at[pl.ds(col, S)], acc)

    def rbody(r, carry):
        pltpu.sync_copy(part_hbm.at[pl.ds(pl.multiple_of(r * _i32(VOCABP) + col, 8), S)], tmp)

        def jbody(j, c2):
            base4 = j * _i32(4 * L)
            for u in range(4):
                sl = pl.ds(base4 + u * L, L)
                acc[sl] = acc[sl] + tmp[sl]
            return c2

        _fori(0, S // (4 * L), jbody, jnp.int32(0))
        return carry

    _fori(1, NW, rbody, jnp.int32(0))
    pltpu.sync_copy(acc, hist_hbm.at[pl.ds(col, S)])


@functools.partial(
    pl.kernel,
    out_type=(
        jax.ShapeDtypeStruct((OUT_PAD,), jnp.int32),
        jax.ShapeDtypeStruct((OUT_PAD,), jnp.int32),
    ),
    mesh=_mesh,
    compiler_params=_params,
    scratch_types=[
        pltpu.VMEM((CCHUNK,), jnp.int32),
        pltpu.VMEM((BUFA,), jnp.int32),
        pltpu.VMEM((BUFA,), jnp.int32),
    ],
)
def _compact_kernel(hist_hbm, val_hbm, cnt_hbm, hchunk, vbuf, cbuf):
    cid = lax.axis_index("c")
    sid = lax.axis_index("s")

    @pl.when(jnp.logical_and(cid == 0, sid == 0))
    def _():
        iota = lax.iota(jnp.int32, L)
        zeros16 = jnp.zeros((L,), jnp.int32)

        def zero_bufs(i, carry):
            base2 = i * _i32(2 * L)
            for u in range(2):
                vbuf[pl.ds(base2 + u * L, L)] = zeros16
                cbuf[pl.ds(base2 + u * L, L)] = zeros16
            return carry

        _fori(0, BUFA // (2 * L), zero_bufs, jnp.int32(0))

        def chunk_body(ci, carry):
            f, g = carry
            pltpu.sync_copy(hist_hbm.at[pl.ds(pl.multiple_of(ci * _i32(CCHUNK), 8), CCHUNK)], hchunk)

            def step(j, sc):
                fc, vals = sc
                for u in range(2):
                    h = hchunk[pl.ds(j * _i32(2 * L) + u * L, L)]
                    m = h > 0
                    plsc.store_compressed(vbuf.at[pl.ds(fc, L)], vals, mask=m)
                    plsc.store_compressed(cbuf.at[pl.ds(fc, L)], h, mask=m)
                    pc = plsc.all_reduce_population_count(m)[0]
                    fc = fc + pc
                    vals = vals + _i32(L)
                return (fc, vals)

            vals0 = ci * _i32(CCHUNK) + iota
            f, _unused = _fori(0, CCHUNK // (2 * L), step, (f, vals0))

            rem = jnp.bitwise_and(f, 7)
            k = f - rem
            tailv = vbuf[pl.ds(k, L)]
            tailc = cbuf[pl.ds(k, L)]
            tailv = jnp.where(iota < rem, tailv, jnp.int32(0))
            tailc = jnp.where(iota < rem, tailc, jnp.int32(0))

            pltpu.sync_copy(vbuf.at[pl.ds(0, BUF)], val_hbm.at[pl.ds(pl.multiple_of(g, 8), BUF)])
            pltpu.sync_copy(cbuf.at[pl.ds(0, BUF)], cnt_hbm.at[pl.ds(pl.multiple_of(g, 8), BUF)])

            _fori(0, BUFA // (2 * L), zero_bufs, jnp.int32(0))
            vbuf[pl.ds(0, L)] = tailv
            cbuf[pl.ds(0, L)] = tailc
            return (rem, g + k)

        f, g = _fori(0, VOCABP // CCHUNK, chunk_body, (jnp.int32(0), jnp.int32(0)))

        pltpu.sync_copy(vbuf.at[pl.ds(0, BUF)], val_hbm.at[pl.ds(pl.multiple_of(g, 8), BUF)])
        pltpu.sync_copy(cbuf.at[pl.ds(0, BUF)], cnt_hbm.at[pl.ds(pl.multiple_of(g, 8), BUF)])

        _fori(0, BUFA // (2 * L), zero_bufs, jnp.int32(0))

        def ztail(i, carry):
            off = jnp.minimum(g + _i32(BUF) + i * _i32(BUF), _i32(OUT_PAD - BUF))
            pltpu.sync_copy(vbuf.at[pl.ds(0, BUF)], val_hbm.at[pl.ds(pl.multiple_of(off, 8), BUF)])
            pltpu.sync_copy(cbuf.at[pl.ds(0, BUF)], cnt_hbm.at[pl.ds(pl.multiple_of(off, 8), BUF)])
            return carry

        _fori(0, MAXZ, ztail, jnp.int32(0))


def kernel(input_tensor, output_tensor, count_tensor):
    del output_tensor, count_tensor
    x = input_tensor.astype(jnp.uint32)
    partials = _hist_kernel(x)
    hist = _merge_kernel(partials)
    values, counts = _compact_kernel(hist)
    return (
        values[:VOCAB].astype(jnp.int64),
        counts[:VOCAB].astype(jnp.int64),
    )
